# baseline (speedup 1.0000x reference)
import jax
import jax.numpy as jnp
from jax.experimental import pallas as pl
from jax.experimental.pallas import tpu as pltpu
from jax.experimental.pallas import tpu_sc as plsc

_BBLK = 8


def kernel(token_ids, matrix):
    b, s = token_ids.shape
    n, d = matrix.shape
    nblocks = b // _BBLK
    half = nblocks // 2
    indices = token_ids.astype(jnp.int32).reshape(nblocks, 1, _BBLK * s)

    mesh = plsc.VectorSubcoreMesh(
        core_axis_name="core", subcore_axis_name="subcore"
    )

    @pl.kernel(
        out_type=jax.ShapeDtypeStruct((b, s, d), matrix.dtype),
        mesh=mesh,
    )
    def gather_kernel(x_hbm, i_hbm, o_hbm):
        core = jax.lax.axis_index("core")

        def body(i_vmem, o_vmem):
            pltpu.sync_copy(
                x_hbm.at[i_vmem.at[0, 0]],
                o_vmem.reshape(_BBLK * s, d),
            )

        pltpu.emit_pipeline(
            body,
            grid=(half,),
            in_specs=[
                pl.BlockSpec(
                    (1, 1, _BBLK * s),
                    index_map=lambda i: (core * half + i, 0, 0),
                )
            ],
            out_specs=[
                pl.BlockSpec(
                    (_BBLK, s, d),
                    index_map=lambda i: (core * half + i, 0, 0),
                )
            ],
            core_axis_name="subcore",
            dimension_semantics=(pltpu.PARALLEL,),
            trace_scopes=False,
        )(i_hbm, o_hbm)

    return gather_kernel(matrix, indices)
